# baseline (speedup 1.0000x reference)
import jax
import jax.numpy as jnp
from jax.experimental import pallas as pl

K_TOPK = 2
ACCEPT_RADIUS = 1.0
CONF_THRESH = 0.025
NUM_HYP = 512
NUM_REFINE = 5


def _apply_transform(points, transform):
    R = transform[..., :3, :3]
    t = transform[..., :3, 3]
    return jnp.matmul(points, jnp.swapaxes(R, -1, -2)) + t[..., None, :]


def _kabsch_rotation(H):
    U, S, Vt = jnp.linalg.svd(H, full_matrices=False)
    V = jnp.swapaxes(Vt, -1, -2)
    Ut = jnp.swapaxes(U, -1, -2)
    det = jnp.linalg.det(jnp.matmul(V, Ut))
    diag = jnp.stack([jnp.ones_like(det), jnp.ones_like(det), det], axis=-1)
    Dm = jnp.eye(3, dtype=H.dtype) * diag[..., None, :]
    return jnp.matmul(jnp.matmul(V, Dm), Ut)


def _weighted_procrustes(src, ref, w):
    w = w / (jnp.sum(w) + 1e-8)
    src_c = jnp.sum(w[:, None] * src, axis=0)
    ref_c = jnp.sum(w[:, None] * ref, axis=0)
    src0 = src - src_c
    ref0 = ref - ref_c
    H = jnp.matmul((src0 * w[:, None]).T, ref0)
    R = _kabsch_rotation(H)
    t = ref_c - jnp.matmul(R, src_c)
    T = jnp.eye(4, dtype=src.dtype).at[:3, :3].set(R).at[:3, 3].set(t)
    return T


def _compute_correspondence_indices(score_mat, ref_knn_masks, src_knn_masks):
    mask_mat = jnp.logical_and(ref_knn_masks[:, :, None], src_knn_masks[:, None, :])
    B, RL, SL = score_mat.shape
    b_idx = jnp.arange(B)[:, None, None]
    st = jnp.swapaxes(score_mat, 1, 2)
    src_topk_scores, src_topk_idx = jax.lax.top_k(st, K_TOPK)
    s_idx = jnp.arange(SL)[None, :, None]
    src_score_mat = jnp.zeros_like(score_mat).at[b_idx, src_topk_idx, s_idx].set(src_topk_scores)
    voter_corr_mat = src_score_mat > CONF_THRESH
    num_corr = min(NUM_HYP, mask_mat.size)
    flat = score_mat.reshape(-1)
    corr_scores, corr_idx = jax.lax.top_k(flat, num_corr)
    bs = corr_idx // (RL * SL)
    rem = corr_idx % (RL * SL)
    rs = rem // SL
    ss = rem % RL
    corr_mat = jnp.zeros(mask_mat.shape, dtype=bool).at[bs, rs, ss].set(True)
    corr_mat = jnp.logical_and(corr_mat, mask_mat)
    voter_corr_mat = jnp.logical_and(voter_corr_mat, mask_mat)
    bi, ri, si = jnp.nonzero(corr_mat, size=num_corr, fill_value=0)
    num_voters = B * SL * K_TOPK
    bv, rv, sv = jnp.nonzero(voter_corr_mat, size=num_voters, fill_value=0)
    vmask = jnp.arange(num_voters) < jnp.sum(voter_corr_mat)
    return bi, ri, si, bv, rv, sv, vmask


def _id_body(x_ref, o_ref):
    o_ref[...] = x_ref[...]


def kernel(ref_knn_points, src_knn_points, re_ref_knn_feats, re_src_knn_feats, ref_knn_masks, src_knn_masks, score_mat):
    bi, ri, si, bv, rv, sv, vmask = _compute_correspondence_indices(score_mat, ref_knn_masks, src_knn_masks)
    vw = vmask.astype(jnp.float32)
    g_ref = ref_knn_points[bi, ri]
    g_src = src_knn_points[bi, si]
    g_scores = score_mat[bi, ri, si]
    ref_cf = re_ref_knn_feats[bi, ri]
    src_cf = re_src_knn_feats[bi, si]
    ref_cp = ref_knn_points[bv, rv]
    src_cp = src_knn_points[bv, sv]
    c_scores = score_mat[bv, rv, sv] * vw
    Rl = _kabsch_rotation(jnp.matmul(jnp.swapaxes(src_cf, -1, -2), ref_cf))
    aligned = jnp.einsum('bmn,bn->bm', Rl, g_src)
    t = g_ref - aligned
    C = Rl.shape[0]
    transforms = jnp.tile(jnp.eye(4, dtype=jnp.float32)[None], (C, 1, 1))
    transforms = transforms.at[:, :3, :3].set(Rl).at[:, :3, 3].set(t)
    batch_aligned = _apply_transform(src_cp[None], transforms)
    resid = jnp.linalg.norm(ref_cp[None] - batch_aligned, axis=2)
    inlier = resid < ACCEPT_RADIUS
    ir = jnp.sum(inlier.astype(jnp.float32) * vw[None, :], axis=1) / jnp.sum(vw)
    best = jnp.argmax(ir)
    cur = c_scores * inlier[best].astype(jnp.float32)
    est = _weighted_procrustes(src_cp, ref_cp, cur)
    for _ in range(NUM_REFINE - 1):
        a2 = _apply_transform(src_cp, est)
        r2 = jnp.linalg.norm(ref_cp - a2, axis=1)
        cur = c_scores * (r2 < ACCEPT_RADIUS).astype(jnp.float32)
        est = _weighted_procrustes(src_cp, ref_cp, cur)

    g_scores = pl.pallas_call(
        _id_body,
        out_shape=jax.ShapeDtypeStruct(g_scores.shape, g_scores.dtype),
    )(g_scores)
    return (g_ref, g_src, g_scores, est, transforms, ref_cf, src_cf)


# speedup vs baseline: 2.6442x; 2.6442x over previous
import jax
import jax.numpy as jnp
from jax import lax
from jax.experimental import pallas as pl
from jax.experimental.pallas import tpu as pltpu

K_TOPK = 2
ACCEPT_RADIUS = 1.0
CONF_THRESH = 0.025
NUM_HYP = 512
NUM_REFINE = 5
N, K, D = 128, 64, 16
NV = N * K * K_TOPK
NSQ = 12


def _quat_terms_from_col(y0, y1, y2, y3):
    n = y0 * y0 + y1 * y1 + y2 * y2 + y3 * y3
    inv = 1.0 / (n + 1e-30)
    ww = y0 * y0 * inv
    wx = y0 * y1 * inv
    wy = y0 * y2 * inv
    wz = y0 * y3 * inv
    xx = y1 * y1 * inv
    xy = y1 * y2 * inv
    xz = y1 * y3 * inv
    yy = y2 * y2 * inv
    yz = y2 * y3 * inv
    zz = y3 * y3 * inv
    r00 = 1.0 - 2.0 * (yy + zz)
    r01 = 2.0 * (xy - wz)
    r02 = 2.0 * (xz + wy)
    r10 = 2.0 * (xy + wz)
    r11 = 1.0 - 2.0 * (xx + zz)
    r12 = 2.0 * (yz - wx)
    r20 = 2.0 * (xz - wy)
    r21 = 2.0 * (yz + wx)
    r22 = 1.0 - 2.0 * (xx + yy)
    return (r00, r01, r02, r10, r11, r12, r20, r21, r22)


def _kmat_entries(H):
    (sxx, sxy, sxz), (syx, syy, syz), (szx, szy, szz) = H
    k00 = sxx + syy + szz
    k01 = syz - szy
    k02 = szx - sxz
    k03 = sxy - syx
    k11 = sxx - syy - szz
    k12 = sxy + syx
    k13 = szx + sxz
    k22 = syy - sxx - szz
    k23 = syz + szy
    k33 = szz - sxx - syy
    fro2 = (sxx * sxx + sxy * sxy + sxz * sxz + syx * syx + syy * syy
            + syz * syz + szx * szx + szy * szy + szz * szz)
    return [[k00, k01, k02, k03],
            [k01, k11, k12, k13],
            [k02, k12, k22, k23],
            [k03, k13, k23, k33]], fro2


def _kabsch_elementwise(H):
    A, fro2 = _kmat_entries(H)
    c = 2.0 * jnp.sqrt(fro2) + 1e-30
    for i in range(4):
        A[i][i] = A[i][i] + c
    for _ in range(NSQ):
        B = [[None] * 4 for _ in range(4)]
        n2 = None
        for i in range(4):
            for j in range(4):
                v = A[i][0] * A[0][j]
                for k in range(1, 4):
                    v = v + A[i][k] * A[k][j]
                B[i][j] = v
                n2 = v * v if n2 is None else n2 + v * v
        inv = lax.rsqrt(n2 + 1e-30)
        A = [[B[i][j] * inv for j in range(4)] for i in range(4)]
    d0, d1, d2, d3 = A[0][0], A[1][1], A[2][2], A[3][3]
    m = jnp.maximum(jnp.maximum(d0, d1), jnp.maximum(d2, d3))
    s0 = d0 >= m
    s1 = jnp.logical_and(d1 >= m, jnp.logical_not(s0))
    s2 = jnp.logical_and(d2 >= m, jnp.logical_not(jnp.logical_or(s0, s1)))
    y = []
    for i in range(4):
        v = jnp.where(s0, A[i][0], jnp.where(s1, A[i][1],
                      jnp.where(s2, A[i][2], A[i][3])))
        y.append(v)
    return _quat_terms_from_col(y[0], y[1], y[2], y[3])


def _hyp_kernel(gr_ref, gs_ref, sf_ref, rf_ref, tout_ref):
    gr = gr_ref[:, :, :]
    gs = gs_ref[:, :, :]
    sf = sf_ref[:, :, :]
    rf = rf_ref[:, :, :]
    H = [[None] * 3 for _ in range(3)]
    for i in range(3):
        for j in range(3):
            v = sf[i] * rf[j]
            for d in range(1, D):
                v = v + sf[3 * d + i] * rf[3 * d + j]
            H[i][j] = v
    R = _kabsch_elementwise(H)
    (r00, r01, r02, r10, r11, r12, r20, r21, r22) = R
    t0 = gr[0] - (r00 * gs[0] + r01 * gs[1] + r02 * gs[2])
    t1 = gr[1] - (r10 * gs[0] + r11 * gs[1] + r12 * gs[2])
    t2 = gr[2] - (r20 * gs[0] + r21 * gs[1] + r22 * gs[2])
    zero = jnp.zeros_like(t0)
    one = jnp.ones_like(t0)
    rows = [r00, r01, r02, t0, r10, r11, r12, t1,
            r20, r21, r22, t2, zero, zero, zero, one]
    for k in range(16):
        tout_ref[k, :, :] = rows[k]


def _refine_kernel(qh_ref, sp_ref, rp_ref, csc_ref, vw_ref, est_ref, cnt_ref):
    f32 = jnp.float32
    HB = 64
    VC = 2048
    NC = NV // VC

    def hb_body(hb, carry):
        qb = qh_ref[pl.ds(hb * 8, 8), :]
        cols = [qb[:, k:k + 1] for k in range(12)]
        (r00, r01, r02, t0, r10, r11, r12, t1, r20, r21, r22, t2) = cols

        def c_body(c, acc):
            base = c * VC
            sx = sp_ref[0:1, pl.ds(base, VC)]
            sy = sp_ref[1:2, pl.ds(base, VC)]
            sz = sp_ref[2:3, pl.ds(base, VC)]
            rx = rp_ref[0:1, pl.ds(base, VC)]
            ry = rp_ref[1:2, pl.ds(base, VC)]
            rz = rp_ref[2:3, pl.ds(base, VC)]
            w = vw_ref[0:1, pl.ds(base, VC)]
            d0 = r00 * sx + r01 * sy + r02 * sz + t0 - rx
            d1 = r10 * sx + r11 * sy + r12 * sz + t1 - ry
            d2 = r20 * sx + r21 * sy + r22 * sz + t2 - rz
            r2 = d0 * d0 + d1 * d1 + d2 * d2
            inl = jnp.where(r2 < ACCEPT_RADIUS * ACCEPT_RADIUS, 1.0, 0.0) * w
            return acc + jnp.sum(inl, axis=1, keepdims=True)

        acc = lax.fori_loop(0, NC, c_body, jnp.zeros((8, 1), f32))
        cnt_ref[pl.ds(hb * 8, 8), :] = acc
        return carry

    lax.fori_loop(0, HB, hb_body, 0)

    cnt = cnt_ref[:, :]
    m = jnp.max(cnt)
    iot = lax.broadcasted_iota(jnp.int32, (NUM_HYP, 1), 0)
    best = jnp.min(jnp.where(cnt >= m, iot, jnp.int32(1 << 30)))

    qh = qh_ref[:, :]
    sel = jnp.where(iot == best, 1.0, 0.0).astype(f32)
    qbest = jnp.sum(qh * sel, axis=0, keepdims=True)
    lane16 = lax.broadcasted_iota(jnp.int32, (1, 16), 1)
    bs = [jnp.sum(jnp.where(lane16 == k, qbest, 0.0)) for k in range(12)]
    (r00, r01, r02, t0, r10, r11, r12, t1, r20, r21, r22, t2) = bs

    sx = sp_ref[0:1, :]
    sy = sp_ref[1:2, :]
    sz = sp_ref[2:3, :]
    rx = rp_ref[0:1, :]
    ry = rp_ref[1:2, :]
    rz = rp_ref[2:3, :]
    csc = csc_ref[0:1, :]

    def residual2(rr, tt):
        (a00, a01, a02, a10, a11, a12, a20, a21, a22) = rr
        (b0, b1, b2) = tt
        d0 = a00 * sx + a01 * sy + a02 * sz + b0 - rx
        d1 = a10 * sx + a11 * sy + a12 * sz + b1 - ry
        d2 = a20 * sx + a21 * sy + a22 * sz + b2 - rz
        return d0 * d0 + d1 * d1 + d2 * d2

    r2b = residual2((r00, r01, r02, r10, r11, r12, r20, r21, r22),
                    (t0, t1, t2))
    cur = csc * jnp.where(r2b < ACCEPT_RADIUS * ACCEPT_RADIUS, 1.0, 0.0)

    def wproc(curw):
        s = jnp.sum(curw)
        w = curw * (1.0 / (s + 1e-8))
        scx = jnp.sum(w * sx)
        scy = jnp.sum(w * sy)
        scz = jnp.sum(w * sz)
        rcx = jnp.sum(w * rx)
        rcy = jnp.sum(w * ry)
        rcz = jnp.sum(w * rz)
        ws = [w * (sx - scx), w * (sy - scy), w * (sz - scz)]
        rd = [rx - rcx, ry - rcy, rz - rcz]
        H = [[jnp.sum(ws[i] * rd[j]) for j in range(3)] for i in range(3)]
        rr = _kabsch_elementwise(H)
        (a00, a01, a02, a10, a11, a12, a20, a21, a22) = rr
        b0 = rcx - (a00 * scx + a01 * scy + a02 * scz)
        b1 = rcy - (a10 * scx + a11 * scy + a12 * scz)
        b2 = rcz - (a20 * scx + a21 * scy + a22 * scz)
        return rr, (b0, b1, b2)

    rr, tt = wproc(cur)
    for _ in range(NUM_REFINE - 1):
        r2 = residual2(rr, tt)
        cur = csc * jnp.where(r2 < ACCEPT_RADIUS * ACCEPT_RADIUS, 1.0, 0.0)
        rr, tt = wproc(cur)

    row4 = lax.broadcasted_iota(jnp.int32, (4, 4), 0)
    col4 = lax.broadcasted_iota(jnp.int32, (4, 4), 1)
    vals = [rr[0], rr[1], rr[2], tt[0],
            rr[3], rr[4], rr[5], tt[1],
            rr[6], rr[7], rr[8], tt[2]]
    est = jnp.zeros((4, 4), f32)
    idx = 0
    for i in range(3):
        for j in range(4):
            mask = jnp.logical_and(row4 == i, col4 == j)
            est = est + jnp.where(mask, vals[idx], 0.0)
            idx += 1
    est = est + jnp.where(jnp.logical_and(row4 == 3, col4 == 3), 1.0, 0.0)
    est_ref[:, :] = est


def _sparse_stage(score_mat):
    B, RL, SL = score_mat.shape
    b_idx = jnp.arange(B)[:, None, None]
    st = jnp.swapaxes(score_mat, 1, 2)
    src_topk_scores, src_topk_idx = jax.lax.top_k(st, K_TOPK)
    s_idx = jnp.arange(SL)[None, :, None]
    src_score_mat = jnp.zeros_like(score_mat).at[
        b_idx, src_topk_idx, s_idx].set(src_topk_scores)
    voter_corr_mat = src_score_mat > CONF_THRESH
    num_corr = min(NUM_HYP, score_mat.size)
    flat = score_mat.reshape(-1)
    _, corr_idx = jax.lax.top_k(flat, num_corr)
    bs = corr_idx // (RL * SL)
    rem = corr_idx % (RL * SL)
    rs = rem // SL
    ss = rem % RL
    corr_mat = jnp.zeros(score_mat.shape, dtype=bool).at[bs, rs, ss].set(True)
    bi, ri, si = jnp.nonzero(corr_mat, size=num_corr, fill_value=0)
    num_voters = B * SL * K_TOPK
    bv, rv, sv = jnp.nonzero(voter_corr_mat, size=num_voters, fill_value=0)
    vmask = jnp.arange(num_voters) < jnp.sum(voter_corr_mat)
    return bi, ri, si, bv, rv, sv, vmask


def kernel(ref_knn_points, src_knn_points, re_ref_knn_feats, re_src_knn_feats,
           ref_knn_masks, src_knn_masks, score_mat):
    f32 = jnp.float32
    bi, ri, si, bv, rv, sv, vmask = _sparse_stage(score_mat)
    vw = vmask.astype(f32)
    g_ref = ref_knn_points[bi, ri]
    g_src = src_knn_points[bi, si]
    g_scores = score_mat[bi, ri, si]
    ref_cf = re_ref_knn_feats[bi, ri]
    src_cf = re_src_knn_feats[bi, si]
    ref_cp = ref_knn_points[bv, rv]
    src_cp = src_knn_points[bv, sv]
    c_scores = score_mat[bv, rv, sv] * vw

    gr_in = g_ref.T.reshape(3, 4, 128)
    gs_in = g_src.T.reshape(3, 4, 128)
    sf_in = src_cf.transpose(1, 2, 0).reshape(48, 4, 128)
    rf_in = ref_cf.transpose(1, 2, 0).reshape(48, 4, 128)

    tout = pl.pallas_call(
        _hyp_kernel,
        out_shape=jax.ShapeDtypeStruct((16, 4, 128), f32),
    )(gr_in, gs_in, sf_in, rf_in)

    transforms = tout.reshape(16, NUM_HYP).T.reshape(NUM_HYP, 4, 4)
    qh = transforms.reshape(NUM_HYP, 16)

    sp_in = src_cp.T
    rp_in = ref_cp.T
    csc_in = c_scores[None, :]
    vw_in = vw[None, :]

    est = pl.pallas_call(
        _refine_kernel,
        out_shape=jax.ShapeDtypeStruct((4, 4), f32),
        scratch_shapes=[pltpu.VMEM((NUM_HYP, 1), f32)],
    )(qh, sp_in, rp_in, csc_in, vw_in)

    return (g_ref, g_src, g_scores, est, transforms, ref_cf, src_cf)


# speedup vs baseline: 2.9338x; 1.1095x over previous
import functools

import jax
import jax.numpy as jnp
from jax import lax
from jax.experimental import pallas as pl
from jax.experimental.pallas import tpu as pltpu
from jax.experimental.pallas import tpu_sc as plsc

K_TOPK = 2
ACCEPT_RADIUS = 1.0
CONF_THRESH = 0.025
NUM_HYP = 512
NUM_REFINE = 5
N, K, D = 128, 64, 16
NV = N * K * K_TOPK
NSQ = 12


def _quat_terms_from_col(y0, y1, y2, y3):
    n = y0 * y0 + y1 * y1 + y2 * y2 + y3 * y3
    inv = 1.0 / (n + 1e-30)
    ww = y0 * y0 * inv
    wx = y0 * y1 * inv
    wy = y0 * y2 * inv
    wz = y0 * y3 * inv
    xx = y1 * y1 * inv
    xy = y1 * y2 * inv
    xz = y1 * y3 * inv
    yy = y2 * y2 * inv
    yz = y2 * y3 * inv
    zz = y3 * y3 * inv
    r00 = 1.0 - 2.0 * (yy + zz)
    r01 = 2.0 * (xy - wz)
    r02 = 2.0 * (xz + wy)
    r10 = 2.0 * (xy + wz)
    r11 = 1.0 - 2.0 * (xx + zz)
    r12 = 2.0 * (yz - wx)
    r20 = 2.0 * (xz - wy)
    r21 = 2.0 * (yz + wx)
    r22 = 1.0 - 2.0 * (xx + yy)
    return (r00, r01, r02, r10, r11, r12, r20, r21, r22)


def _kmat_entries(H):
    (sxx, sxy, sxz), (syx, syy, syz), (szx, szy, szz) = H
    k00 = sxx + syy + szz
    k01 = syz - szy
    k02 = szx - sxz
    k03 = sxy - syx
    k11 = sxx - syy - szz
    k12 = sxy + syx
    k13 = szx + sxz
    k22 = syy - sxx - szz
    k23 = syz + szy
    k33 = szz - sxx - syy
    fro2 = (sxx * sxx + sxy * sxy + sxz * sxz + syx * syx + syy * syy
            + syz * syz + szx * szx + szy * szy + szz * szz)
    return [[k00, k01, k02, k03],
            [k01, k11, k12, k13],
            [k02, k12, k22, k23],
            [k03, k13, k23, k33]], fro2


def _kabsch_elementwise(H):
    A, fro2 = _kmat_entries(H)
    c = 2.0 * jnp.sqrt(fro2) + 1e-30
    for i in range(4):
        A[i][i] = A[i][i] + c
    for _ in range(NSQ):
        B = [[None] * 4 for _ in range(4)]
        n2 = None
        for i in range(4):
            for j in range(4):
                v = A[i][0] * A[0][j]
                for k in range(1, 4):
                    v = v + A[i][k] * A[k][j]
                B[i][j] = v
                n2 = v * v if n2 is None else n2 + v * v
        inv = lax.rsqrt(n2 + 1e-30)
        A = [[B[i][j] * inv for j in range(4)] for i in range(4)]
    d0, d1, d2, d3 = A[0][0], A[1][1], A[2][2], A[3][3]
    m = jnp.maximum(jnp.maximum(d0, d1), jnp.maximum(d2, d3))
    s0 = d0 >= m
    s1 = jnp.logical_and(d1 >= m, jnp.logical_not(s0))
    s2 = jnp.logical_and(d2 >= m, jnp.logical_not(jnp.logical_or(s0, s1)))
    y = []
    for i in range(4):
        v = jnp.where(s0, A[i][0], jnp.where(s1, A[i][1],
                      jnp.where(s2, A[i][2], A[i][3])))
        y.append(v)
    return _quat_terms_from_col(y[0], y[1], y[2], y[3])


def _hyp_kernel(gr_ref, gs_ref, sf_ref, rf_ref, tout_ref):
    gr = gr_ref[:, :, :]
    gs = gs_ref[:, :, :]
    sf = sf_ref[:, :, :]
    rf = rf_ref[:, :, :]
    H = [[None] * 3 for _ in range(3)]
    for i in range(3):
        for j in range(3):
            v = sf[i] * rf[j]
            for d in range(1, D):
                v = v + sf[3 * d + i] * rf[3 * d + j]
            H[i][j] = v
    R = _kabsch_elementwise(H)
    (r00, r01, r02, r10, r11, r12, r20, r21, r22) = R
    t0 = gr[0] - (r00 * gs[0] + r01 * gs[1] + r02 * gs[2])
    t1 = gr[1] - (r10 * gs[0] + r11 * gs[1] + r12 * gs[2])
    t2 = gr[2] - (r20 * gs[0] + r21 * gs[1] + r22 * gs[2])
    zero = jnp.zeros_like(t0)
    one = jnp.ones_like(t0)
    rows = [r00, r01, r02, t0, r10, r11, r12, t1,
            r20, r21, r22, t2, zero, zero, zero, one]
    for k in range(16):
        tout_ref[k, :, :] = rows[k]


def _refine_kernel(qh_ref, sp_ref, rp_ref, csc_ref, vw_ref, est_ref, cnt_ref):
    f32 = jnp.float32
    HB = 64
    VC = 2048
    NC = NV // VC

    def hb_body(hb, carry):
        qb = qh_ref[pl.ds(hb * 8, 8), :]
        cols = [qb[:, k:k + 1] for k in range(12)]
        (r00, r01, r02, t0, r10, r11, r12, t1, r20, r21, r22, t2) = cols

        def c_body(c, acc):
            base = c * VC
            sx = sp_ref[0:1, pl.ds(base, VC)]
            sy = sp_ref[1:2, pl.ds(base, VC)]
            sz = sp_ref[2:3, pl.ds(base, VC)]
            rx = rp_ref[0:1, pl.ds(base, VC)]
            ry = rp_ref[1:2, pl.ds(base, VC)]
            rz = rp_ref[2:3, pl.ds(base, VC)]
            w = vw_ref[0:1, pl.ds(base, VC)]
            d0 = r00 * sx + r01 * sy + r02 * sz + t0 - rx
            d1 = r10 * sx + r11 * sy + r12 * sz + t1 - ry
            d2 = r20 * sx + r21 * sy + r22 * sz + t2 - rz
            r2 = d0 * d0 + d1 * d1 + d2 * d2
            inl = jnp.where(r2 < ACCEPT_RADIUS * ACCEPT_RADIUS, 1.0, 0.0) * w
            return acc + jnp.sum(inl, axis=1, keepdims=True)

        acc = lax.fori_loop(0, NC, c_body, jnp.zeros((8, 1), f32))
        cnt_ref[pl.ds(hb * 8, 8), :] = acc
        return carry

    lax.fori_loop(0, HB, hb_body, 0)

    cnt = cnt_ref[:, :]
    m = jnp.max(cnt)
    iot = lax.broadcasted_iota(jnp.int32, (NUM_HYP, 1), 0)
    best = jnp.min(jnp.where(cnt >= m, iot, jnp.int32(1 << 30)))

    qh = qh_ref[:, :]
    sel = jnp.where(iot == best, 1.0, 0.0).astype(f32)
    qbest = jnp.sum(qh * sel, axis=0, keepdims=True)
    lane16 = lax.broadcasted_iota(jnp.int32, (1, 16), 1)
    bs = [jnp.sum(jnp.where(lane16 == k, qbest, 0.0)) for k in range(12)]
    (r00, r01, r02, t0, r10, r11, r12, t1, r20, r21, r22, t2) = bs

    sx = sp_ref[0:1, :]
    sy = sp_ref[1:2, :]
    sz = sp_ref[2:3, :]
    rx = rp_ref[0:1, :]
    ry = rp_ref[1:2, :]
    rz = rp_ref[2:3, :]
    csc = csc_ref[0:1, :]

    def residual2(rr, tt):
        (a00, a01, a02, a10, a11, a12, a20, a21, a22) = rr
        (b0, b1, b2) = tt
        d0 = a00 * sx + a01 * sy + a02 * sz + b0 - rx
        d1 = a10 * sx + a11 * sy + a12 * sz + b1 - ry
        d2 = a20 * sx + a21 * sy + a22 * sz + b2 - rz
        return d0 * d0 + d1 * d1 + d2 * d2

    r2b = residual2((r00, r01, r02, r10, r11, r12, r20, r21, r22),
                    (t0, t1, t2))
    cur = csc * jnp.where(r2b < ACCEPT_RADIUS * ACCEPT_RADIUS, 1.0, 0.0)

    def wproc(curw):
        s = jnp.sum(curw)
        w = curw * (1.0 / (s + 1e-8))
        scx = jnp.sum(w * sx)
        scy = jnp.sum(w * sy)
        scz = jnp.sum(w * sz)
        rcx = jnp.sum(w * rx)
        rcy = jnp.sum(w * ry)
        rcz = jnp.sum(w * rz)
        ws = [w * (sx - scx), w * (sy - scy), w * (sz - scz)]
        rd = [rx - rcx, ry - rcy, rz - rcz]
        H = [[jnp.sum(ws[i] * rd[j]) for j in range(3)] for i in range(3)]
        rr = _kabsch_elementwise(H)
        (a00, a01, a02, a10, a11, a12, a20, a21, a22) = rr
        b0 = rcx - (a00 * scx + a01 * scy + a02 * scz)
        b1 = rcy - (a10 * scx + a11 * scy + a12 * scz)
        b2 = rcz - (a20 * scx + a21 * scy + a22 * scz)
        return rr, (b0, b1, b2)

    rr, tt = wproc(cur)
    for _ in range(NUM_REFINE - 1):
        r2 = residual2(rr, tt)
        cur = csc * jnp.where(r2 < ACCEPT_RADIUS * ACCEPT_RADIUS, 1.0, 0.0)
        rr, tt = wproc(cur)

    row4 = lax.broadcasted_iota(jnp.int32, (4, 4), 0)
    col4 = lax.broadcasted_iota(jnp.int32, (4, 4), 1)
    vals = [rr[0], rr[1], rr[2], tt[0],
            rr[3], rr[4], rr[5], tt[1],
            rr[6], rr[7], rr[8], tt[2]]
    est = jnp.zeros((4, 4), f32)
    idx = 0
    for i in range(3):
        for j in range(4):
            mask = jnp.logical_and(row4 == i, col4 == j)
            est = est + jnp.where(mask, vals[idx], 0.0)
            idx += 1
    est = est + jnp.where(jnp.logical_and(row4 == 3, col4 == 3), 1.0, 0.0)
    est_ref[:, :] = est


NW = 16
SLAB = N * K * K // NW
BPW = N // NW
VPW = NV // NW


def _voter_sc_kernel(score_hbm, fv_hbm, csc_hbm, vw_hbm,
                     score_v, flagv_v, obuf_i, obuf_f, pub_v,
                     cnts_v, offs_v, stl_i, stl_f, ob2_i, ob2_f, ob2_w,
                     cnt_sh, stagei_sh, stages_sh):
    i32 = jnp.int32
    f32 = jnp.float32
    wid = lax.axis_index("s") + lax.axis_index("c")
    lanes = lax.broadcasted_iota(i32, (16,), 0)
    zf = jnp.zeros((16,), f32)
    zi = jnp.zeros((16,), i32)
    pltpu.sync_copy(score_hbm.at[pl.ds(wid * SLAB, SLAB)], score_v)

    def zbody(k, c):
        flagv_v[pl.ds(k * 16, 16)] = zf
        return c
    lax.fori_loop(0, SLAB // 16, zbody, 0)

    def col_body(cb, c):
        base = (cb // 4) * (K * K) + (cb % 4) * 16
        s_lane = (cb % 4) * 16 + lanes

        def r_body(r, carry):
            m1, i1, m2, i2 = carry
            v = score_v[pl.ds(base + r * K, 16)]
            u1 = v > m1
            u2 = jnp.logical_and(v > m2, jnp.logical_not(u1))
            rvec = zi + r
            m2n = jnp.where(u1, m1, jnp.where(u2, v, m2))
            i2n = jnp.where(u1, i1, jnp.where(u2, rvec, i2))
            return (jnp.where(u1, v, m1), jnp.where(u1, rvec, i1), m2n, i2n)

        neg = zf - 1e30
        m1, i1, m2, i2 = lax.fori_loop(0, K, r_body, (neg, zi, neg, zi))
        boff = (cb // 4) * (K * K)
        idx1 = boff + i1 * K + s_lane
        idx2 = boff + i2 * K + s_lane
        one = zf + 1.0
        plsc.store_scatter(flagv_v, [idx1], one, mask=m1 > CONF_THRESH)
        plsc.store_scatter(flagv_v, [idx2], one, mask=m2 > CONF_THRESH)
        return c

    lax.fori_loop(0, BPW * 4, col_body, 0)

    def cbody(k, cursor):
        fl = flagv_v[pl.ds(k * 16, 16)]
        sc = score_v[pl.ds(k * 16, 16)]
        msk = fl > 0.5
        gidx = wid * SLAB + k * 16 + lanes
        mi = jnp.where(msk, 1, 0)
        pos = cursor + plsc.cumsum(mi) - 1
        plsc.store_scatter(obuf_i, [pos], gidx, mask=msk)
        plsc.store_scatter(obuf_f, [pos], sc, mask=msk)
        return cursor + jnp.sum(mi)

    cnt = lax.fori_loop(0, SLAB // 16, cbody, jnp.int32(0))

    pub_v[...] = zi + cnt
    pltpu.sync_copy(pub_v, cnt_sh.at[pl.ds(wid * 16, 16)])
    pltpu.sync_copy(obuf_i, stagei_sh.at[pl.ds(wid * VPW, VPW)])
    pltpu.sync_copy(obuf_f, stages_sh.at[pl.ds(wid * VPW, VPW)])
    plsc.subcore_barrier()

    pltpu.sync_copy(cnt_sh, cnts_v)
    c0 = plsc.load_gather(cnts_v, [lanes * 16])
    cum0 = plsc.cumsum(c0)
    total = jnp.sum(c0)
    offs_v[pl.ds(0, 16)] = cum0 - c0

    pltpu.sync_copy(stagei_sh, stl_i)
    pltpu.sync_copy(stages_sh, stl_f)

    def obody(k, c):
        jv = wid * VPW + k * 16 + lanes
        valid = jv < total
        u = zi
        for bit in (8, 4, 2, 1):
            t = u + bit
            ot = plsc.load_gather(offs_v, [t])
            u = jnp.where(jnp.logical_and(ot <= jv, t <= 15), t, u)
        ou = plsc.load_gather(offs_v, [u])
        src = jnp.minimum(u * VPW + (jv - ou), NV - 1)
        vi = plsc.load_gather(stl_i, [src], mask=valid)
        vs = plsc.load_gather(stl_f, [src], mask=valid)
        ob2_i[pl.ds(k * 16, 16)] = jnp.where(valid, vi, 0)
        ob2_f[pl.ds(k * 16, 16)] = jnp.where(valid, vs, 0.0)
        ob2_w[pl.ds(k * 16, 16)] = jnp.where(valid, zf + 1.0, zf)
        return c

    lax.fori_loop(0, VPW // 16, obody, 0)
    pltpu.sync_copy(ob2_i, fv_hbm.at[pl.ds(wid * VPW, VPW)])
    pltpu.sync_copy(ob2_f, csc_hbm.at[pl.ds(wid * VPW, VPW)])
    pltpu.sync_copy(ob2_w, vw_hbm.at[pl.ds(wid * VPW, VPW)])


def _voter_sc(score_flat):
    i32 = jnp.int32
    f32 = jnp.float32
    mesh = plsc.VectorSubcoreMesh(core_axis_name="c", subcore_axis_name="s",
                                  num_cores=1)
    fn = functools.partial(
        pl.kernel, mesh=mesh,
        compiler_params=pltpu.CompilerParams(needs_layout_passes=False),
        out_type=[jax.ShapeDtypeStruct((NV,), i32),
                  jax.ShapeDtypeStruct((NV,), f32),
                  jax.ShapeDtypeStruct((NV,), f32)],
        scratch_types=[
            pltpu.VMEM((SLAB,), f32),
            pltpu.VMEM((SLAB,), f32),
            pltpu.VMEM((VPW,), i32),
            pltpu.VMEM((VPW,), f32),
            pltpu.VMEM((16,), i32),
            pltpu.VMEM((NW * 16,), i32),
            pltpu.VMEM((16,), i32),
            pltpu.VMEM((NV,), i32),
            pltpu.VMEM((NV,), f32),
            pltpu.VMEM((VPW,), i32),
            pltpu.VMEM((VPW,), f32),
            pltpu.VMEM((VPW,), f32),
            pltpu.VMEM_SHARED((NW * 16,), i32),
            pltpu.VMEM_SHARED((NV,), i32),
            pltpu.VMEM_SHARED((NV,), f32),
        ])(_voter_sc_kernel)
    return fn(score_flat)


def _hyp_select(score_mat):
    B, RL, SL = score_mat.shape
    num_corr = min(NUM_HYP, score_mat.size)
    flat = score_mat.reshape(-1)
    _, corr_idx = jax.lax.top_k(flat, num_corr)
    bs = corr_idx // (RL * SL)
    rem = corr_idx % (RL * SL)
    rs = rem // SL
    ss = rem % RL
    corr_mat = jnp.zeros(score_mat.shape, dtype=bool).at[bs, rs, ss].set(True)
    bi, ri, si = jnp.nonzero(corr_mat, size=num_corr, fill_value=0)
    return bi, ri, si


def kernel(ref_knn_points, src_knn_points, re_ref_knn_feats, re_src_knn_feats,
           ref_knn_masks, src_knn_masks, score_mat):
    f32 = jnp.float32
    bi, ri, si = _hyp_select(score_mat)
    fv, c_scores, vw = _voter_sc(score_mat.reshape(-1))
    bv = fv // (K * K)
    remv = fv % (K * K)
    rv = remv // K
    sv = remv % K
    g_ref = ref_knn_points[bi, ri]
    g_src = src_knn_points[bi, si]
    g_scores = score_mat[bi, ri, si]
    ref_cf = re_ref_knn_feats[bi, ri]
    src_cf = re_src_knn_feats[bi, si]
    ref_cp = ref_knn_points[bv, rv]
    src_cp = src_knn_points[bv, sv]

    gr_in = g_ref.T.reshape(3, 4, 128)
    gs_in = g_src.T.reshape(3, 4, 128)
    sf_in = src_cf.transpose(1, 2, 0).reshape(48, 4, 128)
    rf_in = ref_cf.transpose(1, 2, 0).reshape(48, 4, 128)

    tout = pl.pallas_call(
        _hyp_kernel,
        out_shape=jax.ShapeDtypeStruct((16, 4, 128), f32),
    )(gr_in, gs_in, sf_in, rf_in)

    transforms = tout.reshape(16, NUM_HYP).T.reshape(NUM_HYP, 4, 4)
    qh = transforms.reshape(NUM_HYP, 16)

    sp_in = src_cp.T
    rp_in = ref_cp.T
    csc_in = c_scores[None, :]
    vw_in = vw[None, :]

    est = pl.pallas_call(
        _refine_kernel,
        out_shape=jax.ShapeDtypeStruct((4, 4), f32),
        scratch_shapes=[pltpu.VMEM((NUM_HYP, 1), f32)],
    )(qh, sp_in, rp_in, csc_in, vw_in)

    return (g_ref, g_src, g_scores, est, transforms, ref_cf, src_cf)
